# Initial kernel scaffold; baseline (speedup 1.0000x reference)
import jax
import jax.numpy as jnp
from jax.experimental import pallas as pl


def kernel(u_node, v_node, negative_nodes, nb_labels, emb_u, emb_com):
    raise NotImplementedError("write your pallas kernel here")


# speedup vs baseline: 3.0430x; 3.0430x over previous
import functools

import numpy as np
import jax
import jax.numpy as jnp
from jax import lax
from jax.experimental import pallas as pl
from jax.experimental.pallas import tpu as pltpu
from jax.experimental.pallas import tpu_sc as plsc

D = 128
K = 64
NNEG = 20
LANES = 16
NC, NS = 2, 16
NW = NC * NS
CB = 128

_GAMMA = float(0.01 * 10.0 ** (-1 * np.log10(0.01) / (80 * 10 * 100000 * 10)))


def _ds16(c):
    return pl.ds(c * LANES, LANES)


@functools.cache
def _sc_gather(B):
    BPW = B // NW
    NCHUNK = BPW // CB
    mesh = plsc.VectorSubcoreMesh(core_axis_name="c", subcore_axis_name="s")

    def body(emb_hbm, uidx_hbm, vidx_hbm, nidx_hbm,
             u_out, v_out, ns_out,
             uidx_v, vidx_v, nidx_v, urows_v, vrows_v, nsum_v,
             nbuf0, nbuf1, sem_uv, sem_n0, sem_n1):
        wid = lax.axis_index("s") * NC + lax.axis_index("c")
        nbufs = (nbuf0, nbuf1)
        nsems = (sem_n0, sem_n1)
        for ci in range(NCHUNK):
            base = wid * BPW + ci * CB
            pltpu.sync_copy(uidx_hbm.at[pl.ds(base, CB)], uidx_v)
            pltpu.sync_copy(vidx_hbm.at[pl.ds(base, CB)], vidx_v)
            pltpu.sync_copy(nidx_hbm.at[pl.ds(base, CB)], nidx_v)
            cp_u = pltpu.async_copy(emb_hbm.at[uidx_v], urows_v, sem_uv)
            cp_v = pltpu.async_copy(emb_hbm.at[vidx_v], vrows_v, sem_uv)
            pltpu.async_copy(emb_hbm.at[nidx_v.at[0]], nbuf0, sem_n0)
            pltpu.async_copy(emb_hbm.at[nidx_v.at[1]], nbuf1, sem_n1)

            def g_body(g, carry):
                for s in range(2):
                    b = g * 2 + s
                    buf = nbufs[s]
                    pltpu.make_async_copy(
                        emb_hbm.at[nidx_v.at[b]], buf, nsems[s]).wait()
                    acc = [buf[0, _ds16(c)] for c in range(D // LANES)]
                    for r in range(1, NNEG):
                        for c in range(D // LANES):
                            acc[c] = acc[c] + buf[r, _ds16(c)]
                    nb = b + 2

                    @pl.when(nb < CB)
                    def _():
                        pltpu.async_copy(emb_hbm.at[nidx_v.at[nb]], buf, nsems[s])

                    for c in range(D // LANES):
                        nsum_v[pl.ds(b * D + c * LANES, LANES)] = acc[c]
                return carry

            lax.fori_loop(0, CB // 2, g_body, 0)
            cp_u.wait()
            cp_v.wait()
            pltpu.sync_copy(urows_v, u_out.at[pl.ds(base, CB)])
            pltpu.sync_copy(vrows_v, v_out.at[pl.ds(base, CB)])
            pltpu.sync_copy(nsum_v, ns_out.at[pl.ds(base * D, CB * D)])

    return pl.kernel(
        body,
        out_type=(
            jax.ShapeDtypeStruct((B, D), jnp.float32),
            jax.ShapeDtypeStruct((B, D), jnp.float32),
            jax.ShapeDtypeStruct((B * D,), jnp.float32),
        ),
        mesh=mesh,
        scratch_types=(
            pltpu.VMEM((CB,), jnp.int32),
            pltpu.VMEM((CB,), jnp.int32),
            pltpu.VMEM((CB, NNEG), jnp.int32),
            pltpu.VMEM((CB, D), jnp.float32),
            pltpu.VMEM((CB, D), jnp.float32),
            pltpu.VMEM((CB * D,), jnp.float32),
            pltpu.VMEM((NNEG, D), jnp.float32),
            pltpu.VMEM((NNEG, D), jnp.float32),
            pltpu.SemaphoreType.DMA,
            pltpu.SemaphoreType.DMA,
            pltpu.SemaphoreType.DMA,
        ),
    )


def _tc_body(u_ref, v_ref, ns_ref, com_ref, loss_ref, cc_ref):
    B = u_ref.shape[0]
    u = u_ref[...]
    v = v_ref[...]
    ns = ns_ref[...]
    pos = jnp.sum(u * v, axis=1)
    neg = -jnp.sum(u * ns, axis=1)
    lsum = jnp.sum(jax.nn.log_sigmoid(pos) + jax.nn.log_sigmoid(neg))
    diff = u - com_ref[0:1, :]
    best = jnp.sqrt(jnp.sum(diff * diff, axis=1))
    besti = jnp.zeros((B,), jnp.int32)
    for k in range(1, K):
        diff = u - com_ref[k:k + 1, :]
        dk = jnp.sqrt(jnp.sum(diff * diff, axis=1))
        better = dk < best
        besti = jnp.where(better, k, besti)
        best = jnp.where(better, dk, best)
    final = -(lsum / B) + _GAMMA * (jnp.sum(best * best) / B)
    loss_ref[...] = final.reshape(1, 1)
    cc_ref[...] = besti


def kernel(u_node, v_node, negative_nodes, nb_labels, emb_u, emb_com):
    B = u_node.shape[0]
    uidx = u_node.reshape(B).astype(jnp.int32)
    vidx = v_node.reshape(B).astype(jnp.int32)
    nidx = negative_nodes.astype(jnp.int32)
    u_rows, v_rows, ns_flat = _sc_gather(B)(emb_u, uidx, vidx, nidx)
    ns_rows = ns_flat.reshape(B, D)
    loss, cc = pl.pallas_call(
        _tc_body,
        out_shape=(
            jax.ShapeDtypeStruct((1, 1), jnp.float32),
            jax.ShapeDtypeStruct((B,), jnp.int32),
        ),
    )(u_rows, v_rows, ns_rows, emb_com)
    return loss[0, 0], cc


# speedup vs baseline: 3.3715x; 1.1079x over previous
import functools

import numpy as np
import jax
import jax.numpy as jnp
from jax import lax
from jax.experimental import pallas as pl
from jax.experimental.pallas import tpu as pltpu
from jax.experimental.pallas import tpu_sc as plsc

D = 128
K = 64
NNEG = 20
LANES = 16
NC, NS = 2, 16
NW = NC * NS
CB = 128

_GAMMA = float(0.01 * 10.0 ** (-1 * np.log10(0.01) / (80 * 10 * 100000 * 10)))


def _ds16(c):
    return pl.ds(c * LANES, LANES)


SB = 4
NGRP = CB // SB


@functools.cache
def _sc_gather(B):
    BPW = B // NW
    NCHUNK = BPW // CB
    mesh = plsc.VectorSubcoreMesh(core_axis_name="c", subcore_axis_name="s")

    def body(emb_hbm, uidx_hbm, vidx_hbm, nidx_hbm,
             u_out, v_out, ns_out,
             uidx_v, vidx_v, nidx_v, urows_v, vrows_v, nsum_v,
             nbuf0, nbuf1, sem_uv, sem_n0, sem_n1):
        wid = lax.axis_index("s") * NC + lax.axis_index("c")
        nbufs = (nbuf0, nbuf1)
        nsems = (sem_n0, sem_n1)

        def chunk_body(ci, carry):
            base = wid * BPW + ci * CB
            pltpu.sync_copy(uidx_hbm.at[pl.ds(base, CB)], uidx_v)
            pltpu.sync_copy(vidx_hbm.at[pl.ds(base, CB)], vidx_v)
            pltpu.sync_copy(nidx_hbm.at[pl.ds(base * NNEG, CB * NNEG)], nidx_v)
            cp_u = pltpu.async_copy(emb_hbm.at[uidx_v], urows_v, sem_uv)
            cp_v = pltpu.async_copy(emb_hbm.at[vidx_v], vrows_v, sem_uv)
            pltpu.async_copy(
                emb_hbm.at[nidx_v.at[pl.ds(0, SB * NNEG)]], nbuf0, sem_n0)
            pltpu.async_copy(
                emb_hbm.at[nidx_v.at[pl.ds(SB * NNEG, SB * NNEG)]], nbuf1, sem_n1)

            def g_body(g2, carry2):
                for s in range(2):
                    g = g2 * 2 + s
                    buf = nbufs[s]
                    pltpu.make_async_copy(
                        emb_hbm.at[nidx_v.at[pl.ds(g * (SB * NNEG), SB * NNEG)]],
                        buf, nsems[s]).wait()
                    for bi in range(SB):
                        r0 = bi * NNEG
                        acc = [buf[r0, _ds16(c)] for c in range(D // LANES)]
                        for r in range(1, NNEG):
                            for c in range(D // LANES):
                                acc[c] = acc[c] + buf[r0 + r, _ds16(c)]
                        for c in range(D // LANES):
                            nsum_v[pl.ds((g * SB + bi) * D + c * LANES, LANES)] = acc[c]
                    ng = g + 2

                    @pl.when(ng < NGRP)
                    def _():
                        pltpu.async_copy(
                            emb_hbm.at[nidx_v.at[pl.ds(ng * (SB * NNEG), SB * NNEG)]],
                            buf, nsems[s])
                return carry2

            lax.fori_loop(0, NGRP // 2, g_body, 0)
            cp_u.wait()
            cp_v.wait()
            pltpu.sync_copy(urows_v, u_out.at[pl.ds(base, CB)])
            pltpu.sync_copy(vrows_v, v_out.at[pl.ds(base, CB)])
            pltpu.sync_copy(nsum_v, ns_out.at[pl.ds(base * D, CB * D)])
            return carry

        lax.fori_loop(0, NCHUNK, chunk_body, 0)

    return pl.kernel(
        body,
        out_type=(
            jax.ShapeDtypeStruct((B, D), jnp.float32),
            jax.ShapeDtypeStruct((B, D), jnp.float32),
            jax.ShapeDtypeStruct((B * D,), jnp.float32),
        ),
        mesh=mesh,
        scratch_types=(
            pltpu.VMEM((CB,), jnp.int32),
            pltpu.VMEM((CB,), jnp.int32),
            pltpu.VMEM((CB * NNEG,), jnp.int32),
            pltpu.VMEM((CB, D), jnp.float32),
            pltpu.VMEM((CB, D), jnp.float32),
            pltpu.VMEM((CB * D,), jnp.float32),
            pltpu.VMEM((SB * NNEG, D), jnp.float32),
            pltpu.VMEM((SB * NNEG, D), jnp.float32),
            pltpu.SemaphoreType.DMA,
            pltpu.SemaphoreType.DMA,
            pltpu.SemaphoreType.DMA,
        ),
    )


def _tc_body(u_ref, v_ref, ns_ref, com_ref, loss_ref, cc_ref):
    B = u_ref.shape[0]
    u = u_ref[...]
    v = v_ref[...]
    ns = ns_ref[...]
    pos = jnp.sum(u * v, axis=1)
    neg = -jnp.sum(u * ns, axis=1)
    lsum = jnp.sum(jax.nn.log_sigmoid(pos) + jax.nn.log_sigmoid(neg))
    diff = u - com_ref[0:1, :]
    best = jnp.sqrt(jnp.sum(diff * diff, axis=1))
    besti = jnp.zeros((B,), jnp.int32)
    for k in range(1, K):
        diff = u - com_ref[k:k + 1, :]
        dk = jnp.sqrt(jnp.sum(diff * diff, axis=1))
        better = dk < best
        besti = jnp.where(better, k, besti)
        best = jnp.where(better, dk, best)
    final = -(lsum / B) + _GAMMA * (jnp.sum(best * best) / B)
    loss_ref[...] = final.reshape(1, 1)
    cc_ref[...] = besti


def kernel(u_node, v_node, negative_nodes, nb_labels, emb_u, emb_com):
    B = u_node.shape[0]
    uidx = u_node.reshape(B).astype(jnp.int32)
    vidx = v_node.reshape(B).astype(jnp.int32)
    nidx = negative_nodes.reshape(B * NNEG).astype(jnp.int32)
    u_rows, v_rows, ns_flat = _sc_gather(B)(emb_u, uidx, vidx, nidx)
    ns_rows = ns_flat.reshape(B, D)
    loss, cc = pl.pallas_call(
        _tc_body,
        out_shape=(
            jax.ShapeDtypeStruct((1, 1), jnp.float32),
            jax.ShapeDtypeStruct((B,), jnp.int32),
        ),
    )(u_rows, v_rows, ns_rows, emb_com)
    return loss[0, 0], cc


# speedup vs baseline: 4.7782x; 1.4172x over previous
import functools

import numpy as np
import jax
import jax.numpy as jnp
from jax import lax
from jax.experimental import pallas as pl
from jax.experimental.pallas import tpu as pltpu
from jax.experimental.pallas import tpu_sc as plsc

D = 128
K = 64
NNEG = 20
LANES = 16
NC, NS = 2, 16
NW = NC * NS
CB = 128

_GAMMA = float(0.01 * 10.0 ** (-1 * np.log10(0.01) / (80 * 10 * 100000 * 10)))


def _ds16(c):
    return pl.ds(c * LANES, LANES)


SB = 4
NGRP = CB // SB


@functools.cache
def _sc_gather(B):
    BPW = B // NW
    NCHUNK = BPW // CB
    mesh = plsc.VectorSubcoreMesh(core_axis_name="c", subcore_axis_name="s")

    def body(emb_hbm, uidx_hbm, vidx_hbm, nidx_hbm,
             u_out, v_out, ns_out,
             uidx_v, vidx_v, nidx_v, urows_v, vrows_v, nsum_v,
             nbuf0, nbuf1, sem_uv, sem_n0, sem_n1):
        wid = lax.axis_index("s") * NC + lax.axis_index("c")
        nbufs = (nbuf0, nbuf1)
        nsems = (sem_n0, sem_n1)

        def chunk_body(ci, carry):
            base = wid * BPW + ci * CB
            pltpu.sync_copy(uidx_hbm.at[pl.ds(base, CB)], uidx_v)
            pltpu.sync_copy(vidx_hbm.at[pl.ds(base, CB)], vidx_v)
            pltpu.sync_copy(nidx_hbm.at[pl.ds(base * NNEG, CB * NNEG)], nidx_v)
            cp_u = pltpu.async_copy(emb_hbm.at[uidx_v], urows_v, sem_uv)
            cp_v = pltpu.async_copy(emb_hbm.at[vidx_v], vrows_v, sem_uv)
            pltpu.async_copy(
                emb_hbm.at[nidx_v.at[pl.ds(0, SB * NNEG)]], nbuf0, sem_n0)
            pltpu.async_copy(
                emb_hbm.at[nidx_v.at[pl.ds(SB * NNEG, SB * NNEG)]], nbuf1, sem_n1)

            def g_body(g2, carry2):
                for s in range(2):
                    g = g2 * 2 + s
                    buf = nbufs[s]
                    pltpu.make_async_copy(
                        emb_hbm.at[nidx_v.at[pl.ds(g * (SB * NNEG), SB * NNEG)]],
                        buf, nsems[s]).wait()
                    for bi in range(SB):
                        r0 = bi * NNEG
                        acc = [buf[r0, _ds16(c)] for c in range(D // LANES)]
                        for r in range(1, NNEG):
                            for c in range(D // LANES):
                                acc[c] = acc[c] + buf[r0 + r, _ds16(c)]
                        for c in range(D // LANES):
                            nsum_v[pl.ds((g * SB + bi) * D + c * LANES, LANES)] = acc[c]
                    ng = g + 2

                    @pl.when(ng < NGRP)
                    def _():
                        pltpu.async_copy(
                            emb_hbm.at[nidx_v.at[pl.ds(ng * (SB * NNEG), SB * NNEG)]],
                            buf, nsems[s])
                return carry2

            lax.fori_loop(0, NGRP // 2, g_body, 0)
            cp_u.wait()
            cp_v.wait()
            pltpu.sync_copy(urows_v, u_out.at[pl.ds(base, CB)])
            pltpu.sync_copy(vrows_v, v_out.at[pl.ds(base, CB)])
            pltpu.sync_copy(nsum_v, ns_out.at[pl.ds(base * D, CB * D)])
            return carry

        lax.fori_loop(0, NCHUNK, chunk_body, 0)

    return pl.kernel(
        body,
        out_type=(
            jax.ShapeDtypeStruct((B, D), jnp.float32),
            jax.ShapeDtypeStruct((B, D), jnp.float32),
            jax.ShapeDtypeStruct((B * D,), jnp.float32),
        ),
        mesh=mesh,
        scratch_types=(
            pltpu.VMEM((CB,), jnp.int32),
            pltpu.VMEM((CB,), jnp.int32),
            pltpu.VMEM((CB * NNEG,), jnp.int32),
            pltpu.VMEM((CB, D), jnp.float32),
            pltpu.VMEM((CB, D), jnp.float32),
            pltpu.VMEM((CB * D,), jnp.float32),
            pltpu.VMEM((SB * NNEG, D), jnp.float32),
            pltpu.VMEM((SB * NNEG, D), jnp.float32),
            pltpu.SemaphoreType.DMA,
            pltpu.SemaphoreType.DMA,
            pltpu.SemaphoreType.DMA,
        ),
    )


def _halve_reduce(x):
    while x.shape[0] > 1:
        h = x.shape[0] // 2
        x = x[:h, :] + x[h:, :]
    return x


def _tc_body(B, ut_ref, vt_ref, nst_ref, comt_ref, loss_ref, cc_ref, acc_ref):
    i = pl.program_id(0)
    blk = ut_ref.shape[1]
    ut = ut_ref[...]
    vt = vt_ref[...]
    nst = nst_ref[...]
    pos = _halve_reduce(ut * vt)
    neg = -_halve_reduce(ut * nst)
    lsum = jnp.sum(jax.nn.log_sigmoid(pos) + jax.nn.log_sigmoid(neg))
    diff = ut - comt_ref[:, 0:1]
    best = jnp.sqrt(_halve_reduce(diff * diff))
    besti = jnp.zeros((1, blk), jnp.int32)
    for k in range(1, K):
        diff = ut - comt_ref[:, k:k + 1]
        dk = jnp.sqrt(_halve_reduce(diff * diff))
        better = dk < best
        besti = jnp.where(better, k, besti)
        best = jnp.where(better, dk, best)
    cc_ref[...] = besti[0]
    min2 = jnp.sum(best * best)

    @pl.when(i == 0)
    def _():
        acc_ref[0] = 0.0
        acc_ref[1] = 0.0

    acc_ref[0] = acc_ref[0] + lsum
    acc_ref[1] = acc_ref[1] + min2

    @pl.when(i == pl.num_programs(0) - 1)
    def _():
        final = -(acc_ref[0] / B) + _GAMMA * (acc_ref[1] / B)
        loss_ref[...] = final.reshape(1, 1)


def kernel(u_node, v_node, negative_nodes, nb_labels, emb_u, emb_com):
    B = u_node.shape[0]
    uidx = u_node.reshape(B).astype(jnp.int32)
    vidx = v_node.reshape(B).astype(jnp.int32)
    nidx = negative_nodes.reshape(B * NNEG).astype(jnp.int32)
    u_rows, v_rows, ns_flat = _sc_gather(B)(emb_u, uidx, vidx, nidx)
    ut = u_rows.T
    vt = v_rows.T
    nst = ns_flat.reshape(B, D).T
    BLK = 2048
    loss, cc = pl.pallas_call(
        functools.partial(_tc_body, B),
        grid=(B // BLK,),
        in_specs=[
            pl.BlockSpec((D, BLK), lambda i: (0, i)),
            pl.BlockSpec((D, BLK), lambda i: (0, i)),
            pl.BlockSpec((D, BLK), lambda i: (0, i)),
            pl.BlockSpec((D, K), lambda i: (0, 0)),
        ],
        out_specs=(
            pl.BlockSpec((1, 1), lambda i: (0, 0)),
            pl.BlockSpec((BLK,), lambda i: (i,)),
        ),
        out_shape=(
            jax.ShapeDtypeStruct((1, 1), jnp.float32),
            jax.ShapeDtypeStruct((B,), jnp.int32),
        ),
        scratch_shapes=[pltpu.SMEM((2,), jnp.float32)],
    )(ut, vt, nst, emb_com.T)
    return loss[0, 0], cc
